# initial kernel scaffold (unmeasured)
import jax
import jax.numpy as jnp
from jax import lax
from jax.experimental import pallas as pl
from jax.experimental.pallas import tpu as pltpu


def kernel(
    x,
):
    def body(*refs):
        pass

    out_shape = jax.ShapeDtypeStruct(..., jnp.float32)
    return pl.pallas_call(body, out_shape=out_shape)(...)


# baseline (device time: 13871 ns/iter reference)
import jax
import jax.numpy as jnp
from jax import lax
from jax.experimental import pallas as pl
from jax.experimental.pallas import tpu as pltpu

N_DEV = 8
BLK = 512


def kernel(x):
    m_per, n = x.shape
    n_blocks = m_per // BLK
    m_global = N_DEV * m_per

    def body(x_ref, out_ref, acc_ref, comm_ref, send_sems, recv_sems):
        step = pl.program_id(0)
        my_pos = lax.axis_index("i")

        @pl.when(step == 0)
        def _():
            acc_ref[...] = jnp.zeros_like(acc_ref)

        acc_ref[...] += jnp.sum(x_ref[...], axis=0, keepdims=True)

        @pl.when(step == n_blocks - 1)
        def _():
            barrier_sem = pltpu.get_barrier_semaphore()
            for k in range(1, N_DEV):
                peer = lax.rem(my_pos + k, N_DEV)
                pl.semaphore_signal(
                    barrier_sem, inc=1,
                    device_id=(peer,), device_id_type=pl.DeviceIdType.MESH,
                )
            pl.semaphore_wait(barrier_sem, N_DEV - 1)

            comm_ref[my_pos] = acc_ref[...]

            sends = []
            for k in range(1, N_DEV):
                peer = lax.rem(my_pos + k, N_DEV)
                rdma = pltpu.make_async_remote_copy(
                    src_ref=comm_ref.at[my_pos],
                    dst_ref=comm_ref.at[my_pos],
                    send_sem=send_sems.at[k - 1],
                    recv_sem=recv_sems.at[my_pos],
                    device_id=(peer,),
                    device_id_type=pl.DeviceIdType.MESH,
                )
                rdma.start()
                sends.append(rdma)

            for k in range(1, N_DEV):
                src = lax.rem(my_pos + k, N_DEV)
                recv = pltpu.make_async_remote_copy(
                    src_ref=comm_ref.at[my_pos],
                    dst_ref=comm_ref.at[src],
                    send_sem=send_sems.at[k - 1],
                    recv_sem=recv_sems.at[src],
                    device_id=(src,),
                    device_id_type=pl.DeviceIdType.MESH,
                )
                recv.wait_recv()

            for rdma in sends:
                rdma.wait_send()

            total = jnp.sum(comm_ref[...], axis=0)
            out_ref[...] = total * (1.0 / m_global)

    return pl.pallas_call(
        body,
        grid=(n_blocks,),
        out_shape=jax.ShapeDtypeStruct((1, n), jnp.float32),
        in_specs=[
            pl.BlockSpec((BLK, n), lambda i: (i, 0), memory_space=pltpu.VMEM),
        ],
        out_specs=pl.BlockSpec((1, n), lambda i: (0, 0), memory_space=pltpu.VMEM),
        scratch_shapes=[
            pltpu.VMEM((1, n), jnp.float32),
            pltpu.VMEM((N_DEV, 1, n), jnp.float32),
            pltpu.SemaphoreType.DMA((N_DEV,)),
            pltpu.SemaphoreType.DMA((N_DEV,)),
        ],
        compiler_params=pltpu.CompilerParams(
            collective_id=0,
            dimension_semantics=("arbitrary",),
        ),
    )(x)


# device time: 13276 ns/iter; 1.0448x vs baseline; 1.0448x over previous
import jax
import jax.numpy as jnp
from jax import lax
from jax.experimental import pallas as pl
from jax.experimental.pallas import tpu as pltpu

N_DEV = 8
BLK = 1024


def kernel(x):
    m_per, n = x.shape
    n_blocks = m_per // BLK
    m_global = N_DEV * m_per

    def body(x_ref, out_ref, acc_ref, comm_ref, send_sems, recv_sems):
        step = pl.program_id(0)
        my_pos = lax.axis_index("i")

        @pl.when(step == 0)
        def _():
            acc_ref[...] = jnp.zeros_like(acc_ref)
            barrier_sem = pltpu.get_barrier_semaphore()
            for k in range(1, N_DEV):
                peer = lax.rem(my_pos + k, N_DEV)
                pl.semaphore_signal(
                    barrier_sem, inc=1,
                    device_id=(peer,), device_id_type=pl.DeviceIdType.MESH,
                )

        acc_ref[...] += jnp.sum(x_ref[...], axis=0, keepdims=True)

        @pl.when(step == n_blocks - 1)
        def _():
            barrier_sem = pltpu.get_barrier_semaphore()
            pl.semaphore_wait(barrier_sem, N_DEV - 1)

            comm_ref[my_pos] = acc_ref[...]

            sends = []
            for k in range(1, N_DEV):
                peer = lax.rem(my_pos + k, N_DEV)
                rdma = pltpu.make_async_remote_copy(
                    src_ref=comm_ref.at[my_pos],
                    dst_ref=comm_ref.at[my_pos],
                    send_sem=send_sems.at[k - 1],
                    recv_sem=recv_sems.at[my_pos],
                    device_id=(peer,),
                    device_id_type=pl.DeviceIdType.MESH,
                )
                rdma.start()
                sends.append(rdma)

            for k in range(1, N_DEV):
                src = lax.rem(my_pos + k, N_DEV)
                recv = pltpu.make_async_remote_copy(
                    src_ref=comm_ref.at[my_pos],
                    dst_ref=comm_ref.at[src],
                    send_sem=send_sems.at[k - 1],
                    recv_sem=recv_sems.at[src],
                    device_id=(src,),
                    device_id_type=pl.DeviceIdType.MESH,
                )
                recv.wait_recv()

            for rdma in sends:
                rdma.wait_send()

            total = jnp.sum(comm_ref[...], axis=0)
            out_ref[...] = total * (1.0 / m_global)

    return pl.pallas_call(
        body,
        grid=(n_blocks,),
        out_shape=jax.ShapeDtypeStruct((1, n), jnp.float32),
        in_specs=[
            pl.BlockSpec((BLK, n), lambda i: (i, 0), memory_space=pltpu.VMEM),
        ],
        out_specs=pl.BlockSpec((1, n), lambda i: (0, 0), memory_space=pltpu.VMEM),
        scratch_shapes=[
            pltpu.VMEM((1, n), jnp.float32),
            pltpu.VMEM((N_DEV, 1, n), jnp.float32),
            pltpu.SemaphoreType.DMA((N_DEV,)),
            pltpu.SemaphoreType.DMA((N_DEV,)),
        ],
        compiler_params=pltpu.CompilerParams(
            collective_id=0,
            dimension_semantics=("arbitrary",),
        ),
    )(x)


# device time: 12758 ns/iter; 1.0872x vs baseline; 1.0406x over previous
import os

import jax
import jax.numpy as jnp
from jax import lax
from jax.experimental import pallas as pl
from jax.experimental.pallas import tpu as pltpu

N_DEV = 8
BLK = 512
NBUF = 8
_SKIP_A2A = os.environ.get("SKIP_A2A") == "1"


def kernel(x):
    m_per, n = x.shape
    n_blocks = m_per // BLK
    m_global = N_DEV * m_per

    def body(x_ref, out_ref, buf_ref, copy_sems, comm_ref, send_sems, recv_sems):
        my_pos = lax.axis_index("i")

        if not _SKIP_A2A:
            barrier_sem = pltpu.get_barrier_semaphore()
            for k in range(1, N_DEV):
                peer = lax.rem(my_pos + k, N_DEV)
                pl.semaphore_signal(
                    barrier_sem, inc=1,
                    device_id=(peer,), device_id_type=pl.DeviceIdType.MESH,
                )

        def chunk_copy(i, slot):
            return pltpu.make_async_copy(
                x_ref.at[pl.ds(i * BLK, BLK), :],
                buf_ref.at[slot],
                copy_sems.at[slot],
            )

        for i in range(n_blocks):
            chunk_copy(i, i % NBUF).start()
        acc = jnp.zeros((1, n), jnp.float32)
        for i in range(n_blocks):
            slot = i % NBUF
            chunk_copy(i, slot).wait()
            acc = acc + jnp.sum(buf_ref[slot], axis=0, keepdims=True)

        if _SKIP_A2A:
            out_ref[...] = acc * (1.0 / m_global)
            return

        barrier_sem = pltpu.get_barrier_semaphore()
        pl.semaphore_wait(barrier_sem, N_DEV - 1)
        comm_ref[my_pos] = acc

        sends = []
        for k in range(1, N_DEV):
            peer = lax.rem(my_pos + k, N_DEV)
            rdma = pltpu.make_async_remote_copy(
                src_ref=comm_ref.at[my_pos],
                dst_ref=comm_ref.at[my_pos],
                send_sem=send_sems.at[k - 1],
                recv_sem=recv_sems.at[my_pos],
                device_id=(peer,),
                device_id_type=pl.DeviceIdType.MESH,
            )
            rdma.start()
            sends.append(rdma)

        for k in range(1, N_DEV):
            src = lax.rem(my_pos + k, N_DEV)
            recv = pltpu.make_async_remote_copy(
                src_ref=comm_ref.at[my_pos],
                dst_ref=comm_ref.at[src],
                send_sem=send_sems.at[k - 1],
                recv_sem=recv_sems.at[src],
                device_id=(src,),
                device_id_type=pl.DeviceIdType.MESH,
            )
            recv.wait_recv()

        for rdma in sends:
            rdma.wait_send()

        total = jnp.sum(comm_ref[...], axis=0)
        out_ref[...] = total * (1.0 / m_global)

    x = pltpu.with_memory_space_constraint(x, pltpu.MemorySpace.HBM)
    return pl.pallas_call(
        body,
        out_shape=jax.ShapeDtypeStruct((1, n), jnp.float32),
        in_specs=[pl.BlockSpec(memory_space=pltpu.MemorySpace.HBM)],
        out_specs=pl.BlockSpec(memory_space=pltpu.VMEM),
        scratch_shapes=[
            pltpu.VMEM((NBUF, BLK, n), jnp.float32),
            pltpu.SemaphoreType.DMA((NBUF,)),
            pltpu.VMEM((N_DEV, 1, n), jnp.float32),
            pltpu.SemaphoreType.DMA((N_DEV,)),
            pltpu.SemaphoreType.DMA((N_DEV,)),
        ],
        compiler_params=pltpu.CompilerParams(
            collective_id=None if _SKIP_A2A else 0,
        ),
    )(x)
